# Initial kernel scaffold; baseline (speedup 1.0000x reference)
import jax
import jax.numpy as jnp
from jax.experimental import pallas as pl


def kernel(x, edge_index, W1_l, b1_l, W1_r, W2_l, b2_l, W2_r):
    raise NotImplementedError("write your pallas kernel here")


# speedup vs baseline: 4.8850x; 4.8850x over previous
import functools

import jax
import jax.numpy as jnp
from jax import lax
from jax.experimental import pallas as pl
from jax.experimental.pallas import tpu as pltpu
from jax.experimental.pallas import tpu_sc as plsc

_NC = 2
_NS = 16
_NW = _NC * _NS
_CW = 16
_CH = 80
_HD = 64


def _dot_t(a, w):
    return lax.dot_general(a, w, (((1,), (1,)), ((), ())),
                           preferred_element_type=jnp.float32)


def _tc_pre(x, wl, wr, b):
    N, D = x.shape
    BLK = 1000
    def body(x_ref, wl_ref, wr_ref, b_ref, xla_ref, xlb_ref, xr_ref):
        xb = x_ref[...]
        xl = _dot_t(xb, wl_ref[...])
        xla_ref[...] = xl[:, :_HD]
        xlb_ref[...] = xl[:, _HD:]
        xr_ref[...] = _dot_t(xb, wr_ref[...]) + b_ref[...]
    return pl.pallas_call(
        body,
        grid=(N // BLK,),
        in_specs=[pl.BlockSpec((BLK, D), lambda i: (i, 0)),
                  pl.BlockSpec((D, D), lambda i: (0, 0)),
                  pl.BlockSpec((D, D), lambda i: (0, 0)),
                  pl.BlockSpec((1, D), lambda i: (0, 0))],
        out_specs=[pl.BlockSpec((BLK, _HD), lambda i: (i, 0)),
                   pl.BlockSpec((BLK, _HD), lambda i: (i, 0)),
                   pl.BlockSpec((BLK, D), lambda i: (i, 0))],
        out_shape=[jax.ShapeDtypeStruct((N, _HD), jnp.float32),
                   jax.ShapeDtypeStruct((N, _HD), jnp.float32),
                   jax.ShapeDtypeStruct((N, D), jnp.float32)],
    )(x, wl, wr, b.reshape(1, D))


def _tc_mid(p, c, xr, wl, wr, b):
    N, D = xr.shape
    BLK = 1000
    def body(p_ref, c_ref, xr_ref, wl_ref, wr_ref, b_ref,
             hla_ref, hlb_ref, hr_ref):
        s = jnp.concatenate([p_ref[0, 0] + p_ref[1, 0],
                             p_ref[0, 1] + p_ref[1, 1]], axis=1)
        cnt = c_ref[0, :, 0:1] + c_ref[1, :, 0:1]
        h = jnp.maximum(s / jnp.maximum(cnt, 1.0) + xr_ref[...], 0.0)
        hl = _dot_t(h, wl_ref[...])
        hla_ref[...] = hl[:, :_HD]
        hlb_ref[...] = hl[:, _HD:]
        hr_ref[...] = _dot_t(h, wr_ref[...]) + b_ref[...]
    return pl.pallas_call(
        body,
        grid=(N // BLK,),
        in_specs=[pl.BlockSpec((_NC, 2, BLK, _HD), lambda i: (0, 0, i, 0)),
                  pl.BlockSpec((_NC, BLK, _CW), lambda i: (0, i, 0)),
                  pl.BlockSpec((BLK, D), lambda i: (i, 0)),
                  pl.BlockSpec((D, D), lambda i: (0, 0)),
                  pl.BlockSpec((D, D), lambda i: (0, 0)),
                  pl.BlockSpec((1, D), lambda i: (0, 0))],
        out_specs=[pl.BlockSpec((BLK, _HD), lambda i: (i, 0)),
                   pl.BlockSpec((BLK, _HD), lambda i: (i, 0)),
                   pl.BlockSpec((BLK, D), lambda i: (i, 0))],
        out_shape=[jax.ShapeDtypeStruct((N, _HD), jnp.float32),
                   jax.ShapeDtypeStruct((N, _HD), jnp.float32),
                   jax.ShapeDtypeStruct((N, D), jnp.float32)],
    )(p, c, xr, wl, wr, b.reshape(1, D))


def _tc_post(p, c, hr):
    N, D = hr.shape
    BLK = 1000
    def body(p_ref, c_ref, hr_ref, o_ref):
        s = jnp.concatenate([p_ref[0, 0] + p_ref[1, 0],
                             p_ref[0, 1] + p_ref[1, 1]], axis=1)
        cnt = c_ref[0, :, 0:1] + c_ref[1, :, 0:1]
        o_ref[...] = s / jnp.maximum(cnt, 1.0) + hr_ref[...]
    return pl.pallas_call(
        body,
        grid=(N // BLK,),
        in_specs=[pl.BlockSpec((_NC, 2, BLK, _HD), lambda i: (0, 0, i, 0)),
                  pl.BlockSpec((_NC, BLK, _CW), lambda i: (0, i, 0)),
                  pl.BlockSpec((BLK, D), lambda i: (i, 0))],
        out_specs=pl.BlockSpec((BLK, D), lambda i: (i, 0)),
        out_shape=jax.ShapeDtypeStruct((N, D), jnp.float32),
    )(p, c, hr)


def _sc_agg(y_halves, src_i, dst_i, zeros_nd, count_aux):
    NP = zeros_nd.shape[0]
    _, NCH, CH = src_i.shape
    RPT = NP // _NS
    with_count = count_aux is not None
    mesh = plsc.VectorSubcoreMesh(core_axis_name="c", subcore_axis_name="s")

    out_type = [jax.ShapeDtypeStruct((_NC, 2, NP, _HD), jnp.float32)]
    scratch = [pltpu.VMEM((NCH, CH), jnp.int32),
               pltpu.VMEM((NCH, CH), jnp.int32),
               pltpu.VMEM((CH, _HD), jnp.float32),
               pltpu.VMEM_SHARED((NP, _HD), jnp.float32)]
    if with_count:
        out_type.append(jax.ShapeDtypeStruct((_NC, NP, _CW), jnp.float32))
        scratch += [pltpu.VMEM((CH, _CW), jnp.float32),
                    pltpu.VMEM_SHARED((NP, _CW), jnp.float32)]

    def body(refs):
        if with_count:
            (ya_h, yb_h, src_h, dst_h, znd_h, znc_h, ones_h,
             out_h, outc_h, srcv, dstv, rows, acc, ones_v, accc) = refs
        else:
            (ya_h, yb_h, src_h, dst_h, znd_h,
             out_h, srcv, dstv, rows, acc) = refs
        cid = lax.axis_index("c")
        sid = lax.axis_index("s")
        wid = cid * _NS + sid
        r0 = sid * RPT
        rs = pl.ds(r0, RPT)
        pltpu.sync_copy(src_h.at[wid], srcv)
        pltpu.sync_copy(dst_h.at[wid], dstv)
        if with_count:
            pltpu.sync_copy(ones_h, ones_v)
            pltpu.sync_copy(znc_h.at[rs], accc.at[rs])
        for half, y_h in enumerate((ya_h, yb_h)):
            pltpu.sync_copy(znd_h.at[rs], acc.at[rs])
            plsc.subcore_barrier()

            @pl.loop(0, NCH)
            def _(j, y_h=y_h, first=(half == 0)):
                pltpu.sync_copy(y_h.at[srcv.at[j]], rows)
                pltpu.sync_copy(rows, acc.at[dstv.at[j]], add=True)
                if with_count and first:
                    pltpu.sync_copy(ones_v, accc.at[dstv.at[j]], add=True)

            plsc.subcore_barrier()
            pltpu.sync_copy(acc.at[rs], out_h.at[cid, half, rs])
            plsc.subcore_barrier()
        if with_count:
            pltpu.sync_copy(accc.at[rs], outc_h.at[cid, rs])

    cp = pltpu.CompilerParams(use_tc_tiling_on_sc=False)
    if with_count:
        zeros_nc, ones_c = count_aux

        @functools.partial(pl.kernel, out_type=out_type, mesh=mesh,
                           scratch_types=scratch, compiler_params=cp)
        def k(*refs):
            body(refs)

        return k(y_halves[0], y_halves[1], src_i, dst_i, zeros_nd,
                 zeros_nc, ones_c)

    @functools.partial(pl.kernel, out_type=out_type[0], mesh=mesh,
                       scratch_types=scratch, compiler_params=cp)
    def k2(*refs):
        body(refs)

    return k2(y_halves[0], y_halves[1], src_i, dst_i, zeros_nd)


def kernel(x, edge_index, W1_l, b1_l, W1_r, W2_l, b2_l, W2_r):
    N, D = x.shape
    E = edge_index.shape[1]
    ei = edge_index.astype(jnp.int32)
    nch = E // (_NW * _CH)
    src_i = ei[0].reshape(_NW, nch, _CH)
    dst_i = ei[1].reshape(_NW, nch, _CH)
    npad = -(-N // (8 * _NS)) * (8 * _NS)
    zeros_nd = jnp.zeros((npad, _HD), jnp.float32)
    zeros_nc = jnp.zeros((npad, _CW), jnp.float32)
    ones_c = jnp.ones((_CH, _CW), jnp.float32)

    xla, xlb, xr = _tc_pre(x, W1_l, W1_r, b1_l)
    p1, c1 = _sc_agg((xla, xlb), src_i, dst_i, zeros_nd,
                     (zeros_nc, ones_c))
    hla, hlb, hr = _tc_mid(p1, c1, xr, W2_l, W2_r, b2_l)
    p2 = _sc_agg((hla, hlb), src_i, dst_i, zeros_nd, None)
    return _tc_post(p2, c1, hr)


# speedup vs baseline: 9.6309x; 1.9715x over previous
import functools

import jax
import jax.numpy as jnp
from jax import lax
from jax.experimental import pallas as pl
from jax.experimental.pallas import tpu as pltpu
from jax.experimental.pallas import tpu_sc as plsc

_NC = 2
_NS = 16
_NW = _NC * _NS
_CW = 16
_CH = 200
_HD = 64


def _dot_t(a, w):
    return lax.dot_general(a, w, (((1,), (1,)), ((), ())),
                           preferred_element_type=jnp.float32)


def _tc_pre(x, wl, wr, b):
    N, D = x.shape
    BLK = 1000
    def body(x_ref, wl_ref, wr_ref, b_ref, xla_ref, xlb_ref, xr_ref):
        xb = x_ref[...]
        xl = _dot_t(xb, wl_ref[...])
        xla_ref[...] = xl[:, :_HD]
        xlb_ref[...] = xl[:, _HD:]
        xr_ref[...] = _dot_t(xb, wr_ref[...]) + b_ref[...]
    return pl.pallas_call(
        body,
        grid=(N // BLK,),
        in_specs=[pl.BlockSpec((BLK, D), lambda i: (i, 0)),
                  pl.BlockSpec((D, D), lambda i: (0, 0)),
                  pl.BlockSpec((D, D), lambda i: (0, 0)),
                  pl.BlockSpec((1, D), lambda i: (0, 0))],
        out_specs=[pl.BlockSpec((BLK, _HD), lambda i: (i, 0)),
                   pl.BlockSpec((BLK, _HD), lambda i: (i, 0)),
                   pl.BlockSpec((BLK, D), lambda i: (i, 0))],
        out_shape=[jax.ShapeDtypeStruct((N, _HD), jnp.float32),
                   jax.ShapeDtypeStruct((N, _HD), jnp.float32),
                   jax.ShapeDtypeStruct((N, D), jnp.float32)],
    )(x, wl, wr, b.reshape(1, D))


def _tc_mid(p, c, xr, wl, wr, b):
    N, D = xr.shape
    BLK = 1000
    def body(p_ref, c_ref, xr_ref, wl_ref, wr_ref, b_ref,
             hla_ref, hlb_ref, hr_ref):
        s = jnp.concatenate([p_ref[0, 0] + p_ref[1, 0],
                             p_ref[0, 1] + p_ref[1, 1]], axis=1)
        cnt = c_ref[0, :, 0:1] + c_ref[1, :, 0:1]
        h = jnp.maximum(s / jnp.maximum(cnt, 1.0) + xr_ref[...], 0.0)
        hl = _dot_t(h, wl_ref[...])
        hla_ref[...] = hl[:, :_HD]
        hlb_ref[...] = hl[:, _HD:]
        hr_ref[...] = _dot_t(h, wr_ref[...]) + b_ref[...]
    return pl.pallas_call(
        body,
        grid=(N // BLK,),
        in_specs=[pl.BlockSpec((_NC, 2, BLK, _HD), lambda i: (0, 0, i, 0)),
                  pl.BlockSpec((_NC, BLK, _CW), lambda i: (0, i, 0)),
                  pl.BlockSpec((BLK, D), lambda i: (i, 0)),
                  pl.BlockSpec((D, D), lambda i: (0, 0)),
                  pl.BlockSpec((D, D), lambda i: (0, 0)),
                  pl.BlockSpec((1, D), lambda i: (0, 0))],
        out_specs=[pl.BlockSpec((BLK, _HD), lambda i: (i, 0)),
                   pl.BlockSpec((BLK, _HD), lambda i: (i, 0)),
                   pl.BlockSpec((BLK, D), lambda i: (i, 0))],
        out_shape=[jax.ShapeDtypeStruct((N, _HD), jnp.float32),
                   jax.ShapeDtypeStruct((N, _HD), jnp.float32),
                   jax.ShapeDtypeStruct((N, D), jnp.float32)],
    )(p, c, xr, wl, wr, b.reshape(1, D))


def _tc_post(p, c, hr):
    N, D = hr.shape
    BLK = 1000
    def body(p_ref, c_ref, hr_ref, o_ref):
        s = jnp.concatenate([p_ref[0, 0] + p_ref[1, 0],
                             p_ref[0, 1] + p_ref[1, 1]], axis=1)
        cnt = c_ref[0, :, 0:1] + c_ref[1, :, 0:1]
        o_ref[...] = s / jnp.maximum(cnt, 1.0) + hr_ref[...]
    return pl.pallas_call(
        body,
        grid=(N // BLK,),
        in_specs=[pl.BlockSpec((_NC, 2, BLK, _HD), lambda i: (0, 0, i, 0)),
                  pl.BlockSpec((_NC, BLK, _CW), lambda i: (0, i, 0)),
                  pl.BlockSpec((BLK, D), lambda i: (i, 0))],
        out_specs=pl.BlockSpec((BLK, D), lambda i: (i, 0)),
        out_shape=jax.ShapeDtypeStruct((N, D), jnp.float32),
    )(p, c, hr)


def _sc_agg(y_halves, src_i, dst_i, zeros_nd, count_aux):
    NP = zeros_nd.shape[0]
    _, NCH, CH = src_i.shape
    RPT = NP // _NS
    with_count = count_aux is not None
    mesh = plsc.VectorSubcoreMesh(core_axis_name="c", subcore_axis_name="s")

    out_type = [jax.ShapeDtypeStruct((_NC, 2, NP, _HD), jnp.float32)]
    scratch = [pltpu.VMEM((NCH, CH), jnp.int32),
               pltpu.VMEM((NCH, CH), jnp.int32),
               pltpu.VMEM((CH, _HD), jnp.float32),
               pltpu.VMEM((CH, _HD), jnp.float32),
               pltpu.SemaphoreType.DMA,
               pltpu.SemaphoreType.DMA,
               pltpu.VMEM_SHARED((NP, _HD), jnp.float32)]
    if with_count:
        out_type.append(jax.ShapeDtypeStruct((_NC, NP, _CW), jnp.float32))
        scratch += [pltpu.VMEM((CH, _CW), jnp.float32),
                    pltpu.VMEM_SHARED((NP, _CW), jnp.float32)]

    def body(refs):
        if with_count:
            (ya_h, yb_h, src_h, dst_h, znd_h, znc_h, ones_h,
             out_h, outc_h, srcv, dstv, rows0, rows1, sem0, sem1,
             acc, ones_v, accc) = refs
        else:
            (ya_h, yb_h, src_h, dst_h, znd_h,
             out_h, srcv, dstv, rows0, rows1, sem0, sem1, acc) = refs
        cid = lax.axis_index("c")
        sid = lax.axis_index("s")
        wid = cid * _NS + sid
        r0 = sid * RPT
        rs = pl.ds(r0, RPT)
        pltpu.sync_copy(src_h.at[wid], srcv)
        pltpu.sync_copy(dst_h.at[wid], dstv)
        if with_count:
            pltpu.sync_copy(ones_h, ones_v)
            pltpu.sync_copy(znc_h.at[rs], accc.at[rs])
        for half, y_h in enumerate((ya_h, yb_h)):
            first = with_count and half == 0
            pltpu.sync_copy(znd_h.at[rs], acc.at[rs])
            plsc.subcore_barrier()
            pltpu.make_async_copy(y_h.at[srcv.at[0]], rows0, sem0).start()
            pltpu.make_async_copy(y_h.at[srcv.at[1]], rows1, sem1).start()

            @pl.loop(0, NCH, step=2)
            def _(j, y_h=y_h, first=first):
                pltpu.make_async_copy(y_h.at[srcv.at[j]], rows0, sem0).wait()
                pltpu.sync_copy(rows0, acc.at[dstv.at[j]], add=True)

                @pl.when(j + 2 < NCH)
                def _():
                    pltpu.make_async_copy(
                        y_h.at[srcv.at[j + 2]], rows0, sem0).start()

                if first:
                    pltpu.sync_copy(ones_v, accc.at[dstv.at[j]], add=True)
                pltpu.make_async_copy(
                    y_h.at[srcv.at[j + 1]], rows1, sem1).wait()
                pltpu.sync_copy(rows1, acc.at[dstv.at[j + 1]], add=True)

                @pl.when(j + 3 < NCH)
                def _():
                    pltpu.make_async_copy(
                        y_h.at[srcv.at[j + 3]], rows1, sem1).start()

                if first:
                    pltpu.sync_copy(ones_v, accc.at[dstv.at[j + 1]],
                                    add=True)

            plsc.subcore_barrier()
            pltpu.sync_copy(acc.at[rs], out_h.at[cid, half, rs])
            plsc.subcore_barrier()
        if with_count:
            pltpu.sync_copy(accc.at[rs], outc_h.at[cid, rs])

    cp = pltpu.CompilerParams(use_tc_tiling_on_sc=False)
    if with_count:
        zeros_nc, ones_c = count_aux

        @functools.partial(pl.kernel, out_type=out_type, mesh=mesh,
                           scratch_types=scratch, compiler_params=cp)
        def k(*refs):
            body(refs)

        return k(y_halves[0], y_halves[1], src_i, dst_i, zeros_nd,
                 zeros_nc, ones_c)

    @functools.partial(pl.kernel, out_type=out_type[0], mesh=mesh,
                       scratch_types=scratch, compiler_params=cp)
    def k2(*refs):
        body(refs)

    return k2(y_halves[0], y_halves[1], src_i, dst_i, zeros_nd)


def kernel(x, edge_index, W1_l, b1_l, W1_r, W2_l, b2_l, W2_r):
    N, D = x.shape
    E = edge_index.shape[1]
    ei = edge_index.astype(jnp.int32)
    nch = E // (_NW * _CH)
    src_i = ei[0].reshape(_NW, nch, _CH)
    dst_i = ei[1].reshape(_NW, nch, _CH)
    npad = -(-N // (8 * _NS)) * (8 * _NS)
    zeros_nd = jnp.zeros((npad, _HD), jnp.float32)
    zeros_nc = jnp.zeros((npad, _CW), jnp.float32)
    ones_c = jnp.ones((_CH, _CW), jnp.float32)

    xla, xlb, xr = _tc_pre(x, W1_l, W1_r, b1_l)
    p1, c1 = _sc_agg((xla, xlb), src_i, dst_i, zeros_nd,
                     (zeros_nc, ones_c))
    hla, hlb, hr = _tc_mid(p1, c1, xr, W2_l, W2_r, b2_l)
    p2 = _sc_agg((hla, hlb), src_i, dst_i, zeros_nd, None)
    return _tc_post(p2, c1, hr)


# speedup vs baseline: 9.6466x; 1.0016x over previous
import functools

import jax
import jax.numpy as jnp
from jax import lax
from jax.experimental import pallas as pl
from jax.experimental.pallas import tpu as pltpu
from jax.experimental.pallas import tpu_sc as plsc

_NC = 2
_NS = 16
_NW = _NC * _NS
_CW = 16
_CH = 200
_HD = 64


def _dot_t(a, w):
    return lax.dot_general(a, w, (((1,), (1,)), ((), ())),
                           preferred_element_type=jnp.float32)


def _tc_lin_halves(x, wl):
    N, D = x.shape
    BLK = 1000
    def body(x_ref, wl_ref, xla_ref, xlb_ref):
        xl = _dot_t(x_ref[...], wl_ref[...])
        xla_ref[...] = xl[:, :_HD]
        xlb_ref[...] = xl[:, _HD:]
    return pl.pallas_call(
        body,
        grid=(N // BLK,),
        in_specs=[pl.BlockSpec((BLK, D), lambda i: (i, 0)),
                  pl.BlockSpec((D, D), lambda i: (0, 0))],
        out_specs=[pl.BlockSpec((BLK, _HD), lambda i: (i, 0)),
                   pl.BlockSpec((BLK, _HD), lambda i: (i, 0))],
        out_shape=[jax.ShapeDtypeStruct((N, _HD), jnp.float32),
                   jax.ShapeDtypeStruct((N, _HD), jnp.float32)],
    )(x, wl)


def _tc_lin_bias(x, wr, b):
    N, D = x.shape
    BLK = 1000
    def body(x_ref, wr_ref, b_ref, xr_ref):
        xr_ref[...] = _dot_t(x_ref[...], wr_ref[...]) + b_ref[...]
    return pl.pallas_call(
        body,
        grid=(N // BLK,),
        in_specs=[pl.BlockSpec((BLK, D), lambda i: (i, 0)),
                  pl.BlockSpec((D, D), lambda i: (0, 0)),
                  pl.BlockSpec((1, D), lambda i: (0, 0))],
        out_specs=pl.BlockSpec((BLK, D), lambda i: (i, 0)),
        out_shape=jax.ShapeDtypeStruct((N, D), jnp.float32),
    )(x, wr, b.reshape(1, D))


def _tc_mid(p, c, xr, wl):
    N, D = xr.shape
    BLK = 1000
    def body(p_ref, c_ref, xr_ref, wl_ref, h_ref, hla_ref, hlb_ref):
        s = jnp.concatenate([p_ref[0, 0] + p_ref[1, 0],
                             p_ref[0, 1] + p_ref[1, 1]], axis=1)
        cnt = c_ref[0, :, 0:1] + c_ref[1, :, 0:1]
        h = jnp.maximum(s / jnp.maximum(cnt, 1.0) + xr_ref[...], 0.0)
        h_ref[...] = h
        hl = _dot_t(h, wl_ref[...])
        hla_ref[...] = hl[:, :_HD]
        hlb_ref[...] = hl[:, _HD:]
    return pl.pallas_call(
        body,
        grid=(N // BLK,),
        in_specs=[pl.BlockSpec((_NC, 2, BLK, _HD), lambda i: (0, 0, i, 0)),
                  pl.BlockSpec((_NC, BLK, _CW), lambda i: (0, i, 0)),
                  pl.BlockSpec((BLK, D), lambda i: (i, 0)),
                  pl.BlockSpec((D, D), lambda i: (0, 0))],
        out_specs=[pl.BlockSpec((BLK, D), lambda i: (i, 0)),
                   pl.BlockSpec((BLK, _HD), lambda i: (i, 0)),
                   pl.BlockSpec((BLK, _HD), lambda i: (i, 0))],
        out_shape=[jax.ShapeDtypeStruct((N, D), jnp.float32),
                   jax.ShapeDtypeStruct((N, _HD), jnp.float32),
                   jax.ShapeDtypeStruct((N, _HD), jnp.float32)],
    )(p, c, xr, wl)


def _tc_post(p, c, hr):
    N, D = hr.shape
    BLK = 1000
    def body(p_ref, c_ref, hr_ref, o_ref):
        s = jnp.concatenate([p_ref[0, 0] + p_ref[1, 0],
                             p_ref[0, 1] + p_ref[1, 1]], axis=1)
        cnt = c_ref[0, :, 0:1] + c_ref[1, :, 0:1]
        o_ref[...] = s / jnp.maximum(cnt, 1.0) + hr_ref[...]
    return pl.pallas_call(
        body,
        grid=(N // BLK,),
        in_specs=[pl.BlockSpec((_NC, 2, BLK, _HD), lambda i: (0, 0, i, 0)),
                  pl.BlockSpec((_NC, BLK, _CW), lambda i: (0, i, 0)),
                  pl.BlockSpec((BLK, D), lambda i: (i, 0))],
        out_specs=pl.BlockSpec((BLK, D), lambda i: (i, 0)),
        out_shape=jax.ShapeDtypeStruct((N, D), jnp.float32),
    )(p, c, hr)


def _sc_agg(y_halves, src_i, dst_i, zeros_nd, count_aux):
    NP = zeros_nd.shape[0]
    _, NCH, CH = src_i.shape
    RPT = NP // _NS
    with_count = count_aux is not None
    mesh = plsc.VectorSubcoreMesh(core_axis_name="c", subcore_axis_name="s")

    out_type = [jax.ShapeDtypeStruct((_NC, 2, NP, _HD), jnp.float32)]
    scratch = [pltpu.VMEM((NCH, CH), jnp.int32),
               pltpu.VMEM((NCH, CH), jnp.int32),
               pltpu.VMEM((CH, _HD), jnp.float32),
               pltpu.VMEM((CH, _HD), jnp.float32),
               pltpu.SemaphoreType.DMA,
               pltpu.SemaphoreType.DMA,
               pltpu.VMEM_SHARED((NP, _HD), jnp.float32)]
    if with_count:
        out_type.append(jax.ShapeDtypeStruct((_NC, NP, _CW), jnp.float32))
        scratch += [pltpu.VMEM((CH, _CW), jnp.float32),
                    pltpu.VMEM_SHARED((NP, _CW), jnp.float32)]

    def body(refs):
        if with_count:
            (ya_h, yb_h, src_h, dst_h, znd_h, znc_h, ones_h,
             out_h, outc_h, srcv, dstv, rows0, rows1, sem0, sem1,
             acc, ones_v, accc) = refs
        else:
            (ya_h, yb_h, src_h, dst_h, znd_h,
             out_h, srcv, dstv, rows0, rows1, sem0, sem1, acc) = refs
        cid = lax.axis_index("c")
        sid = lax.axis_index("s")
        wid = cid * _NS + sid
        r0 = sid * RPT
        rs = pl.ds(r0, RPT)
        pltpu.sync_copy(src_h.at[wid], srcv)
        pltpu.sync_copy(dst_h.at[wid], dstv)
        if with_count:
            pltpu.sync_copy(ones_h, ones_v)
            pltpu.sync_copy(znc_h.at[rs], accc.at[rs])
        for half, y_h in enumerate((ya_h, yb_h)):
            first = with_count and half == 0
            pltpu.sync_copy(znd_h.at[rs], acc.at[rs])
            plsc.subcore_barrier()
            pltpu.make_async_copy(y_h.at[srcv.at[0]], rows0, sem0).start()
            pltpu.make_async_copy(y_h.at[srcv.at[1]], rows1, sem1).start()

            @pl.loop(0, NCH, step=2)
            def _(j, y_h=y_h, first=first):
                pltpu.make_async_copy(y_h.at[srcv.at[j]], rows0, sem0).wait()
                pltpu.sync_copy(rows0, acc.at[dstv.at[j]], add=True)

                @pl.when(j + 2 < NCH)
                def _():
                    pltpu.make_async_copy(
                        y_h.at[srcv.at[j + 2]], rows0, sem0).start()

                if first:
                    pltpu.sync_copy(ones_v, accc.at[dstv.at[j]], add=True)
                pltpu.make_async_copy(
                    y_h.at[srcv.at[j + 1]], rows1, sem1).wait()
                pltpu.sync_copy(rows1, acc.at[dstv.at[j + 1]], add=True)

                @pl.when(j + 3 < NCH)
                def _():
                    pltpu.make_async_copy(
                        y_h.at[srcv.at[j + 3]], rows1, sem1).start()

                if first:
                    pltpu.sync_copy(ones_v, accc.at[dstv.at[j + 1]],
                                    add=True)

            plsc.subcore_barrier()
            pltpu.sync_copy(acc.at[rs], out_h.at[cid, half, rs])
            plsc.subcore_barrier()
        if with_count:
            pltpu.sync_copy(accc.at[rs], outc_h.at[cid, rs])

    cp = pltpu.CompilerParams(use_tc_tiling_on_sc=False)
    if with_count:
        zeros_nc, ones_c = count_aux

        @functools.partial(pl.kernel, out_type=out_type, mesh=mesh,
                           scratch_types=scratch, compiler_params=cp)
        def k(*refs):
            body(refs)

        return k(y_halves[0], y_halves[1], src_i, dst_i, zeros_nd,
                 zeros_nc, ones_c)

    @functools.partial(pl.kernel, out_type=out_type[0], mesh=mesh,
                       scratch_types=scratch, compiler_params=cp)
    def k2(*refs):
        body(refs)

    return k2(y_halves[0], y_halves[1], src_i, dst_i, zeros_nd)


def kernel(x, edge_index, W1_l, b1_l, W1_r, W2_l, b2_l, W2_r):
    N, D = x.shape
    E = edge_index.shape[1]
    ei = edge_index.astype(jnp.int32)
    nch = E // (_NW * _CH)
    src_i = ei[0].reshape(_NW, nch, _CH)
    dst_i = ei[1].reshape(_NW, nch, _CH)
    npad = -(-N // (8 * _NS)) * (8 * _NS)
    zeros_nd = jnp.zeros((npad, _HD), jnp.float32)
    zeros_nc = jnp.zeros((npad, _CW), jnp.float32)
    ones_c = jnp.ones((_CH, _CW), jnp.float32)

    xla, xlb = _tc_lin_halves(x, W1_l)
    p1, c1 = _sc_agg((xla, xlb), src_i, dst_i, zeros_nd,
                     (zeros_nc, ones_c))
    xr = _tc_lin_bias(x, W1_r, b1_l)
    h, hla, hlb = _tc_mid(p1, c1, xr, W2_l)
    p2 = _sc_agg((hla, hlb), src_i, dst_i, zeros_nd, None)
    hr = _tc_lin_bias(h, W2_r, b2_l)
    return _tc_post(p2, c1, hr)


# speedup vs baseline: 9.8361x; 1.0196x over previous
import functools

import jax
import jax.numpy as jnp
from jax import lax
from jax.experimental import pallas as pl
from jax.experimental.pallas import tpu as pltpu
from jax.experimental.pallas import tpu_sc as plsc

_NC = 2
_NS = 16
_NW = _NC * _NS
_CW = 16
_CH = 400
_HD = 64


def _dot_t(a, w):
    return lax.dot_general(a, w, (((1,), (1,)), ((), ())),
                           preferred_element_type=jnp.float32)


def _tc_pre(x, wl, wr, b):
    N, D = x.shape
    BLK = 1000
    def body(x_ref, wl_ref, wr_ref, b_ref, xla_ref, xlb_ref, xr_ref):
        xb = x_ref[...]
        xl = _dot_t(xb, wl_ref[...])
        xla_ref[...] = xl[:, :_HD]
        xlb_ref[...] = xl[:, _HD:]
        xr_ref[...] = _dot_t(xb, wr_ref[...]) + b_ref[...]
    return pl.pallas_call(
        body,
        grid=(N // BLK,),
        in_specs=[pl.BlockSpec((BLK, D), lambda i: (i, 0)),
                  pl.BlockSpec((D, D), lambda i: (0, 0)),
                  pl.BlockSpec((D, D), lambda i: (0, 0)),
                  pl.BlockSpec((1, D), lambda i: (0, 0))],
        out_specs=[pl.BlockSpec((BLK, _HD), lambda i: (i, 0)),
                   pl.BlockSpec((BLK, _HD), lambda i: (i, 0)),
                   pl.BlockSpec((BLK, D), lambda i: (i, 0))],
        out_shape=[jax.ShapeDtypeStruct((N, _HD), jnp.float32),
                   jax.ShapeDtypeStruct((N, _HD), jnp.float32),
                   jax.ShapeDtypeStruct((N, D), jnp.float32)],
    )(x, wl, wr, b.reshape(1, D))


def _tc_mid(p, c, xr, wl, wr, b):
    N, D = xr.shape
    BLK = 1000
    def body(p_ref, c_ref, xr_ref, wl_ref, wr_ref, b_ref,
             hla_ref, hlb_ref, hr_ref):
        s = jnp.concatenate([p_ref[0, 0] + p_ref[1, 0],
                             p_ref[0, 1] + p_ref[1, 1]], axis=1)
        cnt = c_ref[0, :, 0:1] + c_ref[1, :, 0:1]
        h = jnp.maximum(s / jnp.maximum(cnt, 1.0) + xr_ref[...], 0.0)
        hl = _dot_t(h, wl_ref[...])
        hla_ref[...] = hl[:, :_HD]
        hlb_ref[...] = hl[:, _HD:]
        hr_ref[...] = _dot_t(h, wr_ref[...]) + b_ref[...]
    return pl.pallas_call(
        body,
        grid=(N // BLK,),
        in_specs=[pl.BlockSpec((_NC, 2, BLK, _HD), lambda i: (0, 0, i, 0)),
                  pl.BlockSpec((_NC, BLK, _CW), lambda i: (0, i, 0)),
                  pl.BlockSpec((BLK, D), lambda i: (i, 0)),
                  pl.BlockSpec((D, D), lambda i: (0, 0)),
                  pl.BlockSpec((D, D), lambda i: (0, 0)),
                  pl.BlockSpec((1, D), lambda i: (0, 0))],
        out_specs=[pl.BlockSpec((BLK, _HD), lambda i: (i, 0)),
                   pl.BlockSpec((BLK, _HD), lambda i: (i, 0)),
                   pl.BlockSpec((BLK, D), lambda i: (i, 0))],
        out_shape=[jax.ShapeDtypeStruct((N, _HD), jnp.float32),
                   jax.ShapeDtypeStruct((N, _HD), jnp.float32),
                   jax.ShapeDtypeStruct((N, D), jnp.float32)],
    )(p, c, xr, wl, wr, b.reshape(1, D))


def _tc_post(p, c, hr):
    N, D = hr.shape
    BLK = 1000
    def body(p_ref, c_ref, hr_ref, o_ref):
        s = jnp.concatenate([p_ref[0, 0] + p_ref[1, 0],
                             p_ref[0, 1] + p_ref[1, 1]], axis=1)
        cnt = c_ref[0, :, 0:1] + c_ref[1, :, 0:1]
        o_ref[...] = s / jnp.maximum(cnt, 1.0) + hr_ref[...]
    return pl.pallas_call(
        body,
        grid=(N // BLK,),
        in_specs=[pl.BlockSpec((_NC, 2, BLK, _HD), lambda i: (0, 0, i, 0)),
                  pl.BlockSpec((_NC, BLK, _CW), lambda i: (0, i, 0)),
                  pl.BlockSpec((BLK, D), lambda i: (i, 0))],
        out_specs=pl.BlockSpec((BLK, D), lambda i: (i, 0)),
        out_shape=jax.ShapeDtypeStruct((N, D), jnp.float32),
    )(p, c, hr)


def _sc_agg(y_halves, src_i, dst_i, zeros_nd, count_aux):
    NP = zeros_nd.shape[0]
    _, NCH, CH = src_i.shape
    RPT = NP // _NS
    with_count = count_aux is not None
    mesh = plsc.VectorSubcoreMesh(core_axis_name="c", subcore_axis_name="s")
    NE = NCH if NCH % 2 == 0 else NCH - 1

    out_type = [jax.ShapeDtypeStruct((_NC, 2, NP, _HD), jnp.float32)]
    scratch = [pltpu.VMEM((NCH, CH), jnp.int32),
               pltpu.VMEM((NCH, CH), jnp.int32),
               pltpu.VMEM((CH, _HD), jnp.float32),
               pltpu.VMEM((CH, _HD), jnp.float32),
               pltpu.SemaphoreType.DMA,
               pltpu.SemaphoreType.DMA,
               pltpu.VMEM_SHARED((NP, _HD), jnp.float32)]
    if with_count:
        out_type.append(jax.ShapeDtypeStruct((_NC, NP, _CW), jnp.float32))
        scratch += [pltpu.VMEM((CH, _CW), jnp.float32),
                    pltpu.VMEM_SHARED((NP, _CW), jnp.float32)]

    def body(refs):
        if with_count:
            (ya_h, yb_h, src_h, dst_h, znd_h, znc_h, ones_h,
             out_h, outc_h, srcv, dstv, rows0, rows1, sem0, sem1,
             acc, ones_v, accc) = refs
        else:
            (ya_h, yb_h, src_h, dst_h, znd_h,
             out_h, srcv, dstv, rows0, rows1, sem0, sem1, acc) = refs
        cid = lax.axis_index("c")
        sid = lax.axis_index("s")
        wid = cid * _NS + sid
        r0 = sid * RPT
        rs = pl.ds(r0, RPT)
        pltpu.sync_copy(src_h.at[wid], srcv)
        pltpu.sync_copy(dst_h.at[wid], dstv)
        if with_count:
            pltpu.sync_copy(ones_h, ones_v)
            pltpu.sync_copy(znc_h.at[rs], accc.at[rs])
        for half, y_h in enumerate((ya_h, yb_h)):
            first = with_count and half == 0
            pltpu.sync_copy(znd_h.at[rs], acc.at[rs])
            plsc.subcore_barrier()
            pltpu.make_async_copy(y_h.at[srcv.at[0]], rows0, sem0).start()
            pltpu.make_async_copy(y_h.at[srcv.at[1]], rows1, sem1).start()

            @pl.loop(0, NE, step=2)
            def _(j, y_h=y_h, first=first):
                pltpu.make_async_copy(y_h.at[srcv.at[j]], rows0, sem0).wait()
                pltpu.sync_copy(rows0, acc.at[dstv.at[j]], add=True)

                @pl.when(j + 2 < NCH)
                def _():
                    pltpu.make_async_copy(
                        y_h.at[srcv.at[j + 2]], rows0, sem0).start()

                if first:
                    pltpu.sync_copy(ones_v, accc.at[dstv.at[j]], add=True)
                pltpu.make_async_copy(
                    y_h.at[srcv.at[j + 1]], rows1, sem1).wait()
                pltpu.sync_copy(rows1, acc.at[dstv.at[j + 1]], add=True)

                @pl.when(j + 3 < NCH)
                def _():
                    pltpu.make_async_copy(
                        y_h.at[srcv.at[j + 3]], rows1, sem1).start()

                if first:
                    pltpu.sync_copy(ones_v, accc.at[dstv.at[j + 1]],
                                    add=True)

            if NCH % 2:
                j = NCH - 1
                pltpu.make_async_copy(y_h.at[srcv.at[j]], rows0, sem0).wait()
                pltpu.sync_copy(rows0, acc.at[dstv.at[j]], add=True)
                if first:
                    pltpu.sync_copy(ones_v, accc.at[dstv.at[j]], add=True)

            plsc.subcore_barrier()
            pltpu.sync_copy(acc.at[rs], out_h.at[cid, half, rs])
            plsc.subcore_barrier()
        if with_count:
            pltpu.sync_copy(accc.at[rs], outc_h.at[cid, rs])

    cp = pltpu.CompilerParams(use_tc_tiling_on_sc=False)
    if with_count:
        zeros_nc, ones_c = count_aux

        @functools.partial(pl.kernel, out_type=out_type, mesh=mesh,
                           scratch_types=scratch, compiler_params=cp)
        def k(*refs):
            body(refs)

        return k(y_halves[0], y_halves[1], src_i, dst_i, zeros_nd,
                 zeros_nc, ones_c)

    @functools.partial(pl.kernel, out_type=out_type[0], mesh=mesh,
                       scratch_types=scratch, compiler_params=cp)
    def k2(*refs):
        body(refs)

    return k2(y_halves[0], y_halves[1], src_i, dst_i, zeros_nd)


def kernel(x, edge_index, W1_l, b1_l, W1_r, W2_l, b2_l, W2_r):
    N, D = x.shape
    E = edge_index.shape[1]
    ei = edge_index.astype(jnp.int32)
    nch = E // (_NW * _CH)
    src_i = ei[0].reshape(_NW, nch, _CH)
    dst_i = ei[1].reshape(_NW, nch, _CH)
    npad = -(-N // (8 * _NS)) * (8 * _NS)
    zeros_nd = jnp.zeros((npad, _HD), jnp.float32)
    zeros_nc = jnp.zeros((npad, _CW), jnp.float32)
    ones_c = jnp.ones((_CH, _CW), jnp.float32)

    xla, xlb, xr = _tc_pre(x, W1_l, W1_r, b1_l)
    p1, c1 = _sc_agg((xla, xlb), src_i, dst_i, zeros_nd,
                     (zeros_nc, ones_c))
    hla, hlb, hr = _tc_mid(p1, c1, xr, W2_l, W2_r, b2_l)
    p2 = _sc_agg((hla, hlb), src_i, dst_i, zeros_nd, None)
    return _tc_post(p2, c1, hr)


# speedup vs baseline: 11.5990x; 1.1792x over previous
import functools

import jax
import jax.numpy as jnp
from jax import lax
from jax.experimental import pallas as pl
from jax.experimental.pallas import tpu as pltpu
from jax.experimental.pallas import tpu_sc as plsc

_NC = 2
_NS = 16
_NW = _NC * _NS
_CW = 16
_CH = 400
_HD = 64


def _dot_t(a, w):
    return lax.dot_general(a, w, (((1,), (1,)), ((), ())),
                           preferred_element_type=jnp.float32)


def _tc_pre(x, wl, wr, b):
    N, D = x.shape
    BLK = 1000
    def body(x_ref, wl_ref, wr_ref, b_ref, xl_ref, xr_ref):
        xb = x_ref[...]
        xl_ref[...] = _dot_t(xb, wl_ref[...])
        xr_ref[...] = _dot_t(xb, wr_ref[...]) + b_ref[...]
    return pl.pallas_call(
        body,
        grid=(N // BLK,),
        in_specs=[pl.BlockSpec((BLK, D), lambda i: (i, 0)),
                  pl.BlockSpec((D, D), lambda i: (0, 0)),
                  pl.BlockSpec((D, D), lambda i: (0, 0)),
                  pl.BlockSpec((1, D), lambda i: (0, 0))],
        out_specs=[pl.BlockSpec((BLK, D), lambda i: (i, 0)),
                   pl.BlockSpec((BLK, D), lambda i: (i, 0))],
        out_shape=[jax.ShapeDtypeStruct((N, D), jnp.float32),
                   jax.ShapeDtypeStruct((N, D), jnp.float32)],
    )(x, wl, wr, b.reshape(1, D))


def _tc_mid(p, c, xr, wl, wr, b):
    N, D = xr.shape
    BLK = 1000
    def body(p_ref, c_ref, xr_ref, wl_ref, wr_ref, b_ref,
             hl_ref, hr_ref):
        s = p_ref[0] + p_ref[1]
        cnt = c_ref[0, :, 0:1] + c_ref[1, :, 0:1]
        h = jnp.maximum(s / jnp.maximum(cnt, 1.0) + xr_ref[...], 0.0)
        hl_ref[...] = _dot_t(h, wl_ref[...])
        hr_ref[...] = _dot_t(h, wr_ref[...]) + b_ref[...]
    return pl.pallas_call(
        body,
        grid=(N // BLK,),
        in_specs=[pl.BlockSpec((_NC, BLK, D), lambda i: (0, i, 0)),
                  pl.BlockSpec((_NC, BLK, _CW), lambda i: (0, i, 0)),
                  pl.BlockSpec((BLK, D), lambda i: (i, 0)),
                  pl.BlockSpec((D, D), lambda i: (0, 0)),
                  pl.BlockSpec((D, D), lambda i: (0, 0)),
                  pl.BlockSpec((1, D), lambda i: (0, 0))],
        out_specs=[pl.BlockSpec((BLK, D), lambda i: (i, 0)),
                   pl.BlockSpec((BLK, D), lambda i: (i, 0))],
        out_shape=[jax.ShapeDtypeStruct((N, D), jnp.float32),
                   jax.ShapeDtypeStruct((N, D), jnp.float32)],
    )(p, c, xr, wl, wr, b.reshape(1, D))


def _tc_post(p, c, hr):
    N, D = hr.shape
    BLK = 1000
    def body(p_ref, c_ref, hr_ref, o_ref):
        s = p_ref[0] + p_ref[1]
        cnt = c_ref[0, :, 0:1] + c_ref[1, :, 0:1]
        o_ref[...] = s / jnp.maximum(cnt, 1.0) + hr_ref[...]
    return pl.pallas_call(
        body,
        grid=(N // BLK,),
        in_specs=[pl.BlockSpec((_NC, BLK, D), lambda i: (0, i, 0)),
                  pl.BlockSpec((_NC, BLK, _CW), lambda i: (0, i, 0)),
                  pl.BlockSpec((BLK, D), lambda i: (i, 0))],
        out_specs=pl.BlockSpec((BLK, D), lambda i: (i, 0)),
        out_shape=jax.ShapeDtypeStruct((N, D), jnp.float32),
    )(p, c, hr)


def _sc_agg(y2, src2_i, dst_i, zeros_nd, count_aux):
    NP = zeros_nd.shape[0]
    _, NCH, CH = src2_i.shape
    RPT = NP // _NS
    with_count = count_aux is not None
    mesh = plsc.VectorSubcoreMesh(core_axis_name="c", subcore_axis_name="s")
    NE = NCH if NCH % 2 == 0 else NCH - 1

    out_type = [jax.ShapeDtypeStruct((_NC, NP, 2 * _HD), jnp.float32)]
    scratch = [pltpu.VMEM((NCH, CH), jnp.int32),
               pltpu.VMEM((NCH, CH), jnp.int32),
               pltpu.VMEM((CH, _HD), jnp.float32),
               pltpu.VMEM((CH, _HD), jnp.float32),
               pltpu.SemaphoreType.DMA,
               pltpu.SemaphoreType.DMA,
               pltpu.VMEM_SHARED((NP, _HD), jnp.float32)]
    if with_count:
        out_type.append(jax.ShapeDtypeStruct((_NC, NP, _CW), jnp.float32))
        scratch += [pltpu.VMEM((CH, _CW), jnp.float32),
                    pltpu.VMEM_SHARED((NP, _CW), jnp.float32)]

    def body(refs):
        if with_count:
            (y_h, src_h, dst_h, znd_h, znc_h, ones_h,
             out_h, outc_h, srcv, dstv, rows0, rows1, sem0, sem1,
             acc, ones_v, accc) = refs
        else:
            (y_h, src_h, dst_h, znd_h,
             out_h, srcv, dstv, rows0, rows1, sem0, sem1, acc) = refs
        cid = lax.axis_index("c")
        sid = lax.axis_index("s")
        wid = cid * _NS + sid
        r0 = sid * RPT
        rs = pl.ds(r0, RPT)
        pltpu.sync_copy(src_h.at[wid], srcv)
        pltpu.sync_copy(dst_h.at[wid], dstv)
        if with_count:
            pltpu.sync_copy(ones_h, ones_v)
            pltpu.sync_copy(znc_h.at[rs], accc.at[rs])
        for half in (0, 1):
            first = with_count and half == 0
            if half == 1:
                @pl.loop(0, NCH)
                def _(j):
                    @pl.loop(0, CH, step=16)
                    def _(k, j=j):
                        sl = (pl.ds(j, 1), pl.ds(k, 16))
                        srcv.at[sl][...] = srcv.at[sl][...] + 1
            pltpu.sync_copy(znd_h.at[rs], acc.at[rs])
            plsc.subcore_barrier()
            pltpu.make_async_copy(y_h.at[srcv.at[0]], rows0, sem0).start()
            pltpu.make_async_copy(y_h.at[srcv.at[1]], rows1, sem1).start()

            @pl.loop(0, NE, step=2)
            def _(j, first=first):
                pltpu.make_async_copy(y_h.at[srcv.at[j]], rows0, sem0).wait()
                pltpu.sync_copy(rows0, acc.at[dstv.at[j]], add=True)

                @pl.when(j + 2 < NCH)
                def _():
                    pltpu.make_async_copy(
                        y_h.at[srcv.at[j + 2]], rows0, sem0).start()

                if first:
                    pltpu.sync_copy(ones_v, accc.at[dstv.at[j]], add=True)
                pltpu.make_async_copy(
                    y_h.at[srcv.at[j + 1]], rows1, sem1).wait()
                pltpu.sync_copy(rows1, acc.at[dstv.at[j + 1]], add=True)

                @pl.when(j + 3 < NCH)
                def _():
                    pltpu.make_async_copy(
                        y_h.at[srcv.at[j + 3]], rows1, sem1).start()

                if first:
                    pltpu.sync_copy(ones_v, accc.at[dstv.at[j + 1]],
                                    add=True)

            if NCH % 2:
                j = NCH - 1
                pltpu.make_async_copy(y_h.at[srcv.at[j]], rows0, sem0).wait()
                pltpu.sync_copy(rows0, acc.at[dstv.at[j]], add=True)
                if first:
                    pltpu.sync_copy(ones_v, accc.at[dstv.at[j]], add=True)

            plsc.subcore_barrier()
            pltpu.sync_copy(acc.at[rs],
                            out_h.at[cid, rs, pl.ds(half * _HD, _HD)])
            plsc.subcore_barrier()
        if with_count:
            pltpu.sync_copy(accc.at[rs], outc_h.at[cid, rs])

    cp = pltpu.CompilerParams(use_tc_tiling_on_sc=False)
    if with_count:
        zeros_nc, ones_c = count_aux

        @functools.partial(pl.kernel, out_type=out_type, mesh=mesh,
                           scratch_types=scratch, compiler_params=cp)
        def k(*refs):
            body(refs)

        return k(y2, src2_i, dst_i, zeros_nd, zeros_nc, ones_c)

    @functools.partial(pl.kernel, out_type=out_type[0], mesh=mesh,
                       scratch_types=scratch, compiler_params=cp)
    def k2(*refs):
        body(refs)

    return k2(y2, src2_i, dst_i, zeros_nd)


def kernel(x, edge_index, W1_l, b1_l, W1_r, W2_l, b2_l, W2_r):
    N, D = x.shape
    E = edge_index.shape[1]
    ei = edge_index.astype(jnp.int32)
    nch = E // (_NW * _CH)
    src2_i = (ei[0] * 2).reshape(_NW, nch, _CH)
    dst_i = ei[1].reshape(_NW, nch, _CH)
    npad = -(-N // (8 * _NS)) * (8 * _NS)
    zeros_nd = jnp.zeros((npad, _HD), jnp.float32)
    zeros_nc = jnp.zeros((npad, _CW), jnp.float32)
    ones_c = jnp.ones((_CH, _CW), jnp.float32)

    xl, xr = _tc_pre(x, W1_l, W1_r, b1_l)
    p1, c1 = _sc_agg(xl.reshape(2 * N, _HD), src2_i, dst_i, zeros_nd,
                     (zeros_nc, ones_c))
    hl, hr = _tc_mid(p1, c1, xr, W2_l, W2_r, b2_l)
    p2 = _sc_agg(hl.reshape(2 * N, _HD), src2_i, dst_i, zeros_nd, None)
    return _tc_post(p2, c1, hr)


# speedup vs baseline: 11.8667x; 1.0231x over previous
import functools

import jax
import jax.numpy as jnp
from jax import lax
from jax.experimental import pallas as pl
from jax.experimental.pallas import tpu as pltpu
from jax.experimental.pallas import tpu_sc as plsc

_NC = 2
_NS = 16
_NW = _NC * _NS
_CW = 16
_CH = 400
_HD = 64


def _dot_t(a, w):
    return lax.dot_general(a, w, (((1,), (1,)), ((), ())),
                           preferred_element_type=jnp.float32)


def _tc_pre(x, wl, wr, b):
    N, D = x.shape
    BLK = 1000
    def body(x_ref, wl_ref, wr_ref, b_ref, xl_ref, xr_ref):
        xb = x_ref[...]
        xl_ref[...] = _dot_t(xb, wl_ref[...])
        xr_ref[...] = _dot_t(xb, wr_ref[...]) + b_ref[...]
    return pl.pallas_call(
        body,
        grid=(N // BLK,),
        in_specs=[pl.BlockSpec((BLK, D), lambda i: (i, 0)),
                  pl.BlockSpec((D, D), lambda i: (0, 0)),
                  pl.BlockSpec((D, D), lambda i: (0, 0)),
                  pl.BlockSpec((1, D), lambda i: (0, 0))],
        out_specs=[pl.BlockSpec((BLK, D), lambda i: (i, 0)),
                   pl.BlockSpec((BLK, D), lambda i: (i, 0))],
        out_shape=[jax.ShapeDtypeStruct((N, D), jnp.float32),
                   jax.ShapeDtypeStruct((N, D), jnp.float32)],
    )(x, wl, wr, b.reshape(1, D))


def _tc_mid(p, c, xr, wl, wr, b):
    N, D = xr.shape
    BLK = 1000
    def body(p_ref, c_ref, xr_ref, wl_ref, wr_ref, b_ref,
             hl_ref, hr_ref):
        s = p_ref[0] + p_ref[1]
        cnt = c_ref[0, :, 0:1] + c_ref[1, :, 0:1]
        h = jnp.maximum(s / jnp.maximum(cnt, 1.0) + xr_ref[...], 0.0)
        hl_ref[...] = _dot_t(h, wl_ref[...])
        hr_ref[...] = _dot_t(h, wr_ref[...]) + b_ref[...]
    return pl.pallas_call(
        body,
        grid=(N // BLK,),
        in_specs=[pl.BlockSpec((_NC, BLK, D), lambda i: (0, i, 0)),
                  pl.BlockSpec((_NC, BLK, _CW), lambda i: (0, i, 0)),
                  pl.BlockSpec((BLK, D), lambda i: (i, 0)),
                  pl.BlockSpec((D, D), lambda i: (0, 0)),
                  pl.BlockSpec((D, D), lambda i: (0, 0)),
                  pl.BlockSpec((1, D), lambda i: (0, 0))],
        out_specs=[pl.BlockSpec((BLK, D), lambda i: (i, 0)),
                   pl.BlockSpec((BLK, D), lambda i: (i, 0))],
        out_shape=[jax.ShapeDtypeStruct((N, D), jnp.float32),
                   jax.ShapeDtypeStruct((N, D), jnp.float32)],
    )(p, c, xr, wl, wr, b.reshape(1, D))


def _tc_post(p, c, hr):
    N, D = hr.shape
    BLK = 1000
    def body(p_ref, c_ref, hr_ref, o_ref):
        s = p_ref[0] + p_ref[1]
        cnt = c_ref[0, :, 0:1] + c_ref[1, :, 0:1]
        o_ref[...] = s / jnp.maximum(cnt, 1.0) + hr_ref[...]
    return pl.pallas_call(
        body,
        grid=(N // BLK,),
        in_specs=[pl.BlockSpec((_NC, BLK, D), lambda i: (0, i, 0)),
                  pl.BlockSpec((_NC, BLK, _CW), lambda i: (0, i, 0)),
                  pl.BlockSpec((BLK, D), lambda i: (i, 0))],
        out_specs=pl.BlockSpec((BLK, D), lambda i: (i, 0)),
        out_shape=jax.ShapeDtypeStruct((N, D), jnp.float32),
    )(p, c, hr)


def _sc_agg(y2, src2_i, dst_i, zeros_nd):
    NP = zeros_nd.shape[0]
    _, NCH, CH = src2_i.shape
    RPT = NP // _NS
    mesh = plsc.VectorSubcoreMesh(core_axis_name="c", subcore_axis_name="s")
    NE = NCH if NCH % 2 == 0 else NCH - 1

    out_type = jax.ShapeDtypeStruct((_NC, NP, 2 * _HD), jnp.float32)
    scratch = [pltpu.VMEM((NCH, CH), jnp.int32),
               pltpu.VMEM((NCH, CH), jnp.int32),
               pltpu.VMEM((CH, _HD), jnp.float32),
               pltpu.VMEM((CH, _HD), jnp.float32),
               pltpu.SemaphoreType.DMA,
               pltpu.SemaphoreType.DMA,
               pltpu.VMEM_SHARED((NP, _HD), jnp.float32)]

    @functools.partial(pl.kernel, out_type=out_type, mesh=mesh,
                       scratch_types=scratch,
                       compiler_params=pltpu.CompilerParams(
                           use_tc_tiling_on_sc=False))
    def k(y_h, src_h, dst_h, znd_h, out_h,
          srcv, dstv, rows0, rows1, sem0, sem1, acc):
        cid = lax.axis_index("c")
        sid = lax.axis_index("s")
        wid = cid * _NS + sid
        r0 = sid * RPT
        rs = pl.ds(r0, RPT)
        pltpu.sync_copy(src_h.at[wid], srcv)
        pltpu.sync_copy(dst_h.at[wid], dstv)
        for half in (0, 1):
            if half == 1:
                @pl.loop(0, NCH)
                def _(j):
                    @pl.loop(0, CH, step=16)
                    def _(k, j=j):
                        sl = (pl.ds(j, 1), pl.ds(k, 16))
                        srcv.at[sl][...] = srcv.at[sl][...] + 1
            pltpu.sync_copy(znd_h.at[rs], acc.at[rs])
            plsc.subcore_barrier()
            pltpu.make_async_copy(y_h.at[srcv.at[0]], rows0, sem0).start()
            pltpu.make_async_copy(y_h.at[srcv.at[1]], rows1, sem1).start()

            @pl.loop(0, NE, step=2)
            def _(j):
                pltpu.make_async_copy(y_h.at[srcv.at[j]], rows0, sem0).wait()
                pltpu.sync_copy(rows0, acc.at[dstv.at[j]], add=True)

                @pl.when(j + 2 < NCH)
                def _():
                    pltpu.make_async_copy(
                        y_h.at[srcv.at[j + 2]], rows0, sem0).start()

                pltpu.make_async_copy(
                    y_h.at[srcv.at[j + 1]], rows1, sem1).wait()
                pltpu.sync_copy(rows1, acc.at[dstv.at[j + 1]], add=True)

                @pl.when(j + 3 < NCH)
                def _():
                    pltpu.make_async_copy(
                        y_h.at[srcv.at[j + 3]], rows1, sem1).start()

            if NCH % 2:
                j = NCH - 1
                pltpu.make_async_copy(y_h.at[srcv.at[j]], rows0, sem0).wait()
                pltpu.sync_copy(rows0, acc.at[dstv.at[j]], add=True)

            plsc.subcore_barrier()
            pltpu.sync_copy(acc.at[rs],
                            out_h.at[cid, rs, pl.ds(half * _HD, _HD)])
            plsc.subcore_barrier()

    return k(y2, src2_i, dst_i, zeros_nd)


def _sc_count(dst_i, zeros_nc, ones_c):
    NP = zeros_nc.shape[0]
    _, NCH, CH = dst_i.shape
    RPT = NP // _NS
    mesh = plsc.VectorSubcoreMesh(core_axis_name="c", subcore_axis_name="s")

    @functools.partial(
        pl.kernel,
        out_type=jax.ShapeDtypeStruct((_NC, NP, _CW), jnp.float32),
        mesh=mesh,
        scratch_types=[pltpu.VMEM((NCH, CH), jnp.int32),
                       pltpu.VMEM((CH, _CW), jnp.float32),
                       pltpu.VMEM_SHARED((NP, _CW), jnp.float32)],
        compiler_params=pltpu.CompilerParams(use_tc_tiling_on_sc=False))
    def k(dst_h, znc_h, ones_h, outc_h, dstv, ones_v, accc):
        cid = lax.axis_index("c")
        sid = lax.axis_index("s")
        wid = cid * _NS + sid
        r0 = sid * RPT
        rs = pl.ds(r0, RPT)
        pltpu.sync_copy(dst_h.at[wid], dstv)
        pltpu.sync_copy(ones_h, ones_v)
        pltpu.sync_copy(znc_h.at[rs], accc.at[rs])
        plsc.subcore_barrier()

        @pl.loop(0, NCH)
        def _(j):
            pltpu.sync_copy(ones_v, accc.at[dstv.at[j]], add=True)

        plsc.subcore_barrier()
        pltpu.sync_copy(accc.at[rs], outc_h.at[cid, rs])

    return k(dst_i, zeros_nc, ones_c)


def kernel(x, edge_index, W1_l, b1_l, W1_r, W2_l, b2_l, W2_r):
    N, D = x.shape
    E = edge_index.shape[1]
    ei = edge_index.astype(jnp.int32)
    nch = E // (_NW * _CH)
    src2_i = (ei[0] * 2).reshape(_NW, nch, _CH)
    dst_i = ei[1].reshape(_NW, nch, _CH)
    npad = -(-N // (8 * _NS)) * (8 * _NS)
    zeros_nd = jnp.zeros((npad, _HD), jnp.float32)
    zeros_nc = jnp.zeros((npad, _CW), jnp.float32)
    ones_c = jnp.ones((_CH, _CW), jnp.float32)

    c1 = _sc_count(dst_i, zeros_nc, ones_c)
    xl, xr = _tc_pre(x, W1_l, W1_r, b1_l)
    p1 = _sc_agg(xl.reshape(2 * N, _HD), src2_i, dst_i, zeros_nd)
    hl, hr = _tc_mid(p1, c1, xr, W2_l, W2_r, b2_l)
    p2 = _sc_agg(hl.reshape(2 * N, _HD), src2_i, dst_i, zeros_nd)
    return _tc_post(p2, c1, hr)
